# baseline (speedup 1.0000x reference)
import functools

import jax
import jax.numpy as jnp
from jax import lax
from jax.experimental import pallas as pl
from jax.experimental.pallas import tpu as pltpu
from jax.experimental.pallas import tpu_sc as plsc

N = 10000
E = 320000
NG = 64

NC = 2
NS = 16
NW = NC * NS
IW = 128
KR = 8
CHUNK = KR * IW
E_PAD = 327680
ITERS = E_PAD // (NW * CHUNK)
ROWS_PW = E_PAD // IW // NW
PAD_ROWS = 128
N_ACC = N + PAD_ROWS
SB = 2

RPT = 624
TAIL = N - NS * RPT


def _sc_degree(dst2d, ones16):
    mesh = plsc.VectorSubcoreMesh(core_axis_name="c", subcore_axis_name="s")

    @functools.partial(
        pl.kernel,
        out_type=jax.ShapeDtypeStruct((2, N, 16), jnp.float32),
        mesh=mesh,
        scratch_types=[
            pltpu.VMEM((KR, IW), jnp.int32),
            pltpu.VMEM((IW, 16), jnp.float32),
            pltpu.VMEM_SHARED((N_ACC, 16), jnp.float32),
        ],
        compiler_params=pltpu.CompilerParams(use_tc_tiling_on_sc=False),
    )
    def deg_kernel(dst_h, ones_h, out_h, dstv, onesv, acc):
        cid = lax.axis_index("c")
        sid = lax.axis_index("s")
        wid = sid * NC + cid
        ibase = pl.multiple_of(sid * RPT, 8)
        pltpu.sync_copy(ones_h.at[pl.ds(ibase, RPT)], acc.at[pl.ds(ibase, RPT)])

        @pl.when(sid == NS - 1)
        def _():
            tb = NS * RPT
            pltpu.sync_copy(ones_h.at[pl.ds(tb, TAIL)], acc.at[pl.ds(tb, TAIL)])

        pltpu.sync_copy(ones_h.at[pl.ds(0, IW)], onesv)
        plsc.subcore_barrier()

        def body(t, carry):
            rb = pl.multiple_of(wid * ROWS_PW + t * KR, 8)
            pltpu.sync_copy(dst_h.at[pl.ds(rb, KR)], dstv)
            for j in range(KR):
                pltpu.sync_copy(onesv, acc.at[dstv.at[j]], add=True)
            return carry

        lax.fori_loop(0, ITERS, body, 0)
        plsc.subcore_barrier()
        pltpu.sync_copy(acc.at[pl.ds(ibase, RPT)],
                        out_h.at[cid, pl.ds(ibase, RPT)])

        @pl.when(sid == NS - 1)
        def _():
            tb = NS * RPT
            pltpu.sync_copy(acc.at[pl.ds(tb, TAIL)],
                            out_h.at[cid, pl.ds(tb, TAIL)])

    return deg_kernel(dst2d, ones16)


def _edge_pipeline(src_h, dst_h, table, acc, srcv, dstv, rows, gsems, ssems,
                   isems, row_base, iters, sb):
    def fire_idx(t, slot):
        rb = pl.multiple_of(row_base(t), 8)
        pltpu.async_copy(src_h.at[pl.ds(rb, KR)],
                         srcv.at[pl.ds(slot * KR, KR)], isems[2 * slot])
        pltpu.async_copy(dst_h.at[pl.ds(rb, KR)],
                         dstv.at[pl.ds(slot * KR, KR)], isems[2 * slot + 1])

    def wait_idx(t, slot):
        rb = pl.multiple_of(row_base(t), 8)
        pltpu.make_async_copy(src_h.at[pl.ds(rb, KR)],
                              srcv.at[pl.ds(slot * KR, KR)],
                              isems[2 * slot]).wait()
        pltpu.make_async_copy(dst_h.at[pl.ds(rb, KR)],
                              dstv.at[pl.ds(slot * KR, KR)],
                              isems[2 * slot + 1]).wait()

    def process(slot):
        base = slot * KR
        gh, sh = {}, {}

        def fire_gather(j):
            s = j % sb
            gh[j] = pltpu.async_copy(table.at[srcv.at[base + j]],
                                     rows.at[pl.ds(s * IW, IW)], gsems[s])

        def fire_scatter(j):
            s = j % sb
            sh[j] = pltpu.async_copy(rows.at[pl.ds(s * IW, IW)],
                                     acc.at[dstv.at[base + j]], ssems[s],
                                     add=True)

        for k in range(min(sb, KR)):
            fire_gather(k)
        for j in range(KR):
            gh[j].wait()
            fire_scatter(j)
            k = j + sb
            if k < KR:
                sh[j].wait()
                fire_gather(k)
        for j in range(max(0, KR - sb), KR):
            sh[j].wait()

    half = iters // 2
    fire_idx(0, 0)

    def body(t2, carry):
        ta = 2 * t2
        wait_idx(ta, 0)
        fire_idx(ta + 1, 1)
        process(0)
        wait_idx(ta + 1, 1)

        @pl.when(t2 + 1 < half)
        def _():
            fire_idx(ta + 2, 0)

        process(1)
        return carry

    lax.fori_loop(0, half, body, 0)


def _sc_scatter(hp, src2d, dst2d):
    F = hp.shape[1]
    table_in_spmem = F <= 64
    sb = 5 if table_in_spmem else SB
    mesh = plsc.VectorSubcoreMesh(core_axis_name="c", subcore_axis_name="s")

    scratch = [
        pltpu.VMEM((2 * KR, IW), jnp.int32),
        pltpu.VMEM((2 * KR, IW), jnp.int32),
        pltpu.VMEM((sb * IW, F), jnp.float32),
        pltpu.VMEM_SHARED((N_ACC, F), jnp.float32),
    ] + [pltpu.SemaphoreType.DMA] * (2 * sb + 4)
    if table_in_spmem:
        scratch.append(pltpu.VMEM_SHARED((N, F), jnp.float32))

    @functools.partial(
        pl.kernel,
        out_type=jax.ShapeDtypeStruct((2, N, F), jnp.float32),
        mesh=mesh,
        scratch_types=scratch,
        compiler_params=pltpu.CompilerParams(use_tc_tiling_on_sc=False),
    )
    def scatter_kernel(hp_h, src_h, dst_h, out_h, srcv, dstv, rows, acc,
                       *rest):
        cid = lax.axis_index("c")
        sid = lax.axis_index("s")
        wid = sid * NC + cid
        gsems = rest[:sb]
        ssems = rest[sb:2 * sb]
        isems = rest[2 * sb:2 * sb + 4]
        table = rest[2 * sb + 4] if table_in_spmem else hp_h
        ibase = pl.multiple_of(sid * RPT, 8)
        pltpu.sync_copy(hp_h.at[pl.ds(ibase, RPT)], acc.at[pl.ds(ibase, RPT)])
        if table_in_spmem:
            pltpu.sync_copy(hp_h.at[pl.ds(ibase, RPT)],
                            table.at[pl.ds(ibase, RPT)])

        @pl.when(sid == NS - 1)
        def _():
            tb = NS * RPT
            pltpu.sync_copy(hp_h.at[pl.ds(tb, TAIL)], acc.at[pl.ds(tb, TAIL)])
            if table_in_spmem:
                pltpu.sync_copy(hp_h.at[pl.ds(tb, TAIL)],
                                table.at[pl.ds(tb, TAIL)])

        plsc.subcore_barrier()

        _edge_pipeline(src_h, dst_h, table, acc, srcv, dstv, rows,
                       gsems, ssems, isems,
                       lambda t: wid * ROWS_PW + t * KR, ITERS, sb)
        plsc.subcore_barrier()
        pltpu.sync_copy(acc.at[pl.ds(ibase, RPT)],
                        out_h.at[cid, pl.ds(ibase, RPT)])

        @pl.when(sid == NS - 1)
        def _():
            tb = NS * RPT
            pltpu.sync_copy(acc.at[pl.ds(tb, TAIL)],
                            out_h.at[cid, pl.ds(tb, TAIL)])

    return scatter_kernel(hp, src2d, dst2d)


def _sc_scatter_panels(hpa, hpb, src2d, dst2d):
    HH = hpa.shape[1]
    sb = 5
    rows_pt = (E_PAD // IW) // NS
    iters = rows_pt // KR
    mesh = plsc.VectorSubcoreMesh(core_axis_name="c", subcore_axis_name="s")

    scratch = [
        pltpu.VMEM((2 * KR, IW), jnp.int32),
        pltpu.VMEM((2 * KR, IW), jnp.int32),
        pltpu.VMEM((sb * IW, HH), jnp.float32),
        pltpu.VMEM_SHARED((N_ACC, HH), jnp.float32),
        pltpu.VMEM_SHARED((N, HH), jnp.float32),
    ] + [pltpu.SemaphoreType.DMA] * (2 * sb + 4)

    @functools.partial(
        pl.kernel,
        out_type=jax.ShapeDtypeStruct((2, N, HH), jnp.float32),
        mesh=mesh,
        scratch_types=scratch,
        compiler_params=pltpu.CompilerParams(use_tc_tiling_on_sc=False),
    )
    def panel_kernel(hpa_h, hpb_h, src_h, dst_h, out_h, srcv, dstv, rows,
                     acc, table, *sems):
        cid = lax.axis_index("c")
        sid = lax.axis_index("s")
        gsems = sems[:sb]
        ssems = sems[sb:2 * sb]
        isems = sems[2 * sb:]
        ibase = pl.multiple_of(sid * RPT, 8)

        def init_from(hp_h):
            pltpu.sync_copy(hp_h.at[pl.ds(ibase, RPT)],
                            acc.at[pl.ds(ibase, RPT)])
            pltpu.sync_copy(hp_h.at[pl.ds(ibase, RPT)],
                            table.at[pl.ds(ibase, RPT)])

            @pl.when(sid == NS - 1)
            def _():
                tb = NS * RPT
                pltpu.sync_copy(hp_h.at[pl.ds(tb, TAIL)],
                                acc.at[pl.ds(tb, TAIL)])
                pltpu.sync_copy(hp_h.at[pl.ds(tb, TAIL)],
                                table.at[pl.ds(tb, TAIL)])

        @pl.when(cid == 0)
        def _():
            init_from(hpa_h)

        @pl.when(cid == 1)
        def _():
            init_from(hpb_h)

        plsc.subcore_barrier()

        _edge_pipeline(src_h, dst_h, table, acc, srcv, dstv, rows,
                       gsems, ssems, isems,
                       lambda t: sid * rows_pt + t * KR, iters, sb)
        plsc.subcore_barrier()
        pltpu.sync_copy(acc.at[pl.ds(ibase, RPT)],
                        out_h.at[cid, pl.ds(ibase, RPT)])

        @pl.when(sid == NS - 1)
        def _():
            tb = NS * RPT
            pltpu.sync_copy(acc.at[pl.ds(tb, TAIL)],
                            out_h.at[cid, pl.ds(tb, TAIL)])

    return panel_kernel(hpa, hpb, src2d, dst2d)


_RB = 1000


def _dis_from(deg_blk):
    deg = deg_blk[0, :, 0:1] + deg_blk[1, :, 0:1] - 1.0
    return lax.rsqrt(deg)


def _prep1_body(x_ref, w_ref, deg_ref, out_ref):
    dis = _dis_from(deg_ref[...])
    out_ref[...] = dis * jnp.dot(x_ref[...], w_ref[...],
                                 preferred_element_type=jnp.float32)


def _tc_prep1(x, W1, degout):
    DIN, H = W1.shape
    return pl.pallas_call(
        _prep1_body,
        grid=(N // _RB,),
        in_specs=[
            pl.BlockSpec((_RB, DIN), lambda i: (i, 0)),
            pl.BlockSpec((DIN, H), lambda i: (0, 0)),
            pl.BlockSpec((2, _RB, 16), lambda i: (0, i, 0)),
        ],
        out_specs=pl.BlockSpec((_RB, H), lambda i: (i, 0)),
        out_shape=jax.ShapeDtypeStruct((N, H), jnp.float32),
    )(x, W1, degout)


def _mid_body(p_ref, hp1_ref, deg_ref, b1_ref, w2_ref, outa_ref, outb_ref):
    dis = _dis_from(deg_ref[...])
    s = p_ref[0] + p_ref[1] - hp1_ref[...]
    h1 = jnp.maximum(dis * s + b1_ref[...], 0.0)
    hp2 = dis * jnp.dot(h1, w2_ref[...],
                        preferred_element_type=jnp.float32)
    hh = hp2.shape[1] // 2
    outa_ref[...] = hp2[:, :hh]
    outb_ref[...] = hp2[:, hh:]


def _tc_mid(p, hp1, degout, b1r, W2):
    H1, H2 = W2.shape
    HH = H2 // 2
    return pl.pallas_call(
        _mid_body,
        grid=(N // _RB,),
        in_specs=[
            pl.BlockSpec((2, _RB, H1), lambda i: (0, i, 0)),
            pl.BlockSpec((_RB, H1), lambda i: (i, 0)),
            pl.BlockSpec((2, _RB, 16), lambda i: (0, i, 0)),
            pl.BlockSpec((1, H1), lambda i: (0, 0)),
            pl.BlockSpec((H1, H2), lambda i: (0, 0)),
        ],
        out_specs=[
            pl.BlockSpec((_RB, HH), lambda i: (i, 0)),
            pl.BlockSpec((_RB, HH), lambda i: (i, 0)),
        ],
        out_shape=[
            jax.ShapeDtypeStruct((N, HH), jnp.float32),
            jax.ShapeDtypeStruct((N, HH), jnp.float32),
        ],
    )(p, hp1, degout, b1r, W2)


def _final_body(q_ref, deg_ref, b2_ref,
                batch_ref, wf_ref, bf_ref,
                wm1_ref, bm1_ref, wm2_ref, bm2_ref, wm3_ref, bm3_ref,
                wm4_ref, bm4_ref, y_ref, out_ref, loss_ref, sums_acc, cnt_acc):
    i = pl.program_id(0)

    @pl.when(i == 0)
    def _():
        sums_acc[...] = jnp.zeros_like(sums_acc)
        cnt_acc[...] = jnp.zeros_like(cnt_acc)

    dis = _dis_from(deg_ref[...])
    hh = q_ref.shape[2]
    h2a = jnp.maximum(dis * q_ref[0] + b2_ref[:, :hh], 0.0)
    h2b = jnp.maximum(dis * q_ref[1] + b2_ref[:, hh:], 0.0)
    h2 = jnp.concatenate([h2a, h2b], axis=1)
    b = batch_ref[...]
    onehot = (b == lax.broadcasted_iota(jnp.int32, (_RB, NG), 1)
              ).astype(jnp.float32)
    dn = (((0,), (0,)), ((), ()))
    sums_acc[...] += lax.dot_general(onehot, h2, dn,
                                     preferred_element_type=jnp.float32)
    cnt_acc[...] += lax.dot_general(onehot, jnp.ones((_RB, 1), jnp.float32),
                                    dn, preferred_element_type=jnp.float32)

    @pl.when(i == pl.num_programs(0) - 1)
    def _():
        pool = sums_acc[...] / jnp.maximum(cnt_acc[...], 1.0)
        g = jnp.dot(pool, wf_ref[...],
                    preferred_element_type=jnp.float32) + bf_ref[...]
        m = jnp.maximum(jnp.dot(g, wm1_ref[...],
                                preferred_element_type=jnp.float32)
                        + bm1_ref[...], 0.0)
        m = jnp.maximum(jnp.dot(m, wm2_ref[...],
                                preferred_element_type=jnp.float32)
                        + bm2_ref[...], 0.0)
        m = jnp.maximum(jnp.dot(m, wm3_ref[...],
                                preferred_element_type=jnp.float32)
                        + bm3_ref[...], 0.0)
        o = jnp.dot(m, wm4_ref[...],
                    preferred_element_type=jnp.float32) + bm4_ref[...]
        out_ref[...] = o
        loss_ref[...] = jnp.mean((o - y_ref[...]) ** 2).reshape(1, 1)


def _tc_final(q, degout, b2r, batch2d, Wf, bfr,
              Wm1, bm1r, Wm2, bm2r, Wm3, bm3r, Wm4, bm4r, y):
    HH = q.shape[2]
    H2 = 2 * HH
    D = Wf.shape[1]

    def full(shp):
        nd = len(shp)
        return pl.BlockSpec(shp, lambda i, _n=nd: (0,) * _n)

    return pl.pallas_call(
        _final_body,
        grid=(N // _RB,),
        in_specs=[
            pl.BlockSpec((2, _RB, HH), lambda i: (0, i, 0)),
            pl.BlockSpec((2, _RB, 16), lambda i: (0, i, 0)),
            full((1, H2)),
            pl.BlockSpec((_RB, 1), lambda i: (i, 0)),
            full((H2, D)),
            full((1, D)),
            full((D, 32)), full((1, 32)),
            full((32, 16)), full((1, 16)),
            full((16, 8)), full((1, 8)),
            full((8, 1)), full((1, 1)),
            full((NG, 1)),
        ],
        out_specs=[
            pl.BlockSpec((NG, 1), lambda i: (0, 0)),
            pl.BlockSpec((1, 1), lambda i: (0, 0)),
        ],
        out_shape=[
            jax.ShapeDtypeStruct((NG, 1), jnp.float32),
            jax.ShapeDtypeStruct((1, 1), jnp.float32),
        ],
        scratch_shapes=[
            pltpu.VMEM((NG, H2), jnp.float32),
            pltpu.VMEM((NG, 1), jnp.float32),
        ],
    )(q, degout, b2r, batch2d, Wf, bfr, Wm1, bm1r, Wm2, bm2r,
      Wm3, bm3r, Wm4, bm4r, y)


def kernel(x, edge_index, batch, y, W1, b1, W2, b2, Wf, bf,
           Wm1, bm1, Wm2, bm2, Wm3, bm3, Wm4, bm4):
    npad = E_PAD - E
    src_pad = jnp.concatenate(
        [edge_index[0], jnp.zeros((npad,), jnp.int32)])
    dst_pad = jnp.concatenate(
        [edge_index[1], N + (jnp.arange(npad, dtype=jnp.int32) % PAD_ROWS)])
    src2d = src_pad.reshape(E_PAD // IW, IW)
    dst2d = dst_pad.reshape(E_PAD // IW, IW)
    ones16 = jnp.ones((N, 16), jnp.float32)

    degout = _sc_degree(dst2d, ones16)
    hp1 = _tc_prep1(x, W1, degout)
    p = _sc_scatter(hp1, src2d, dst2d)
    hp2a, hp2b = _tc_mid(p, hp1, degout, b1.reshape(1, -1), W2)
    q = _sc_scatter_panels(hp2a, hp2b, src2d, dst2d)
    out, loss = _tc_final(
        q, degout, b2.reshape(1, -1), batch.reshape(-1, 1),
        Wf, bf.reshape(1, -1), Wm1, bm1.reshape(1, -1), Wm2,
        bm2.reshape(1, -1), Wm3, bm3.reshape(1, -1), Wm4,
        bm4.reshape(1, -1), y)
    return out, loss.reshape(())


# speedup vs baseline: 1.0810x; 1.0810x over previous
import functools

import jax
import jax.numpy as jnp
from jax import lax
from jax.experimental import pallas as pl
from jax.experimental.pallas import tpu as pltpu
from jax.experimental.pallas import tpu_sc as plsc

N = 10000
E = 320000
NG = 64

NC = 2
NS = 16
NW = NC * NS
IW = 128
KR = 8
CHUNK = KR * IW
E_PAD = 327680
ITERS = E_PAD // (NW * CHUNK)
ROWS_PW = E_PAD // IW // NW
PAD_ROWS = 128
N_ACC = N + PAD_ROWS
SB = 2

RPT = 624
TAIL = N - NS * RPT


def _sc_degree(dst2d, ones16):
    mesh = plsc.VectorSubcoreMesh(core_axis_name="c", subcore_axis_name="s")

    @functools.partial(
        pl.kernel,
        out_type=jax.ShapeDtypeStruct((2, N, 16), jnp.float32),
        mesh=mesh,
        scratch_types=[
            pltpu.VMEM((KR, IW), jnp.int32),
            pltpu.VMEM((IW, 16), jnp.float32),
            pltpu.VMEM_SHARED((N_ACC, 16), jnp.float32),
        ],
        compiler_params=pltpu.CompilerParams(use_tc_tiling_on_sc=False),
    )
    def deg_kernel(dst_h, ones_h, out_h, dstv, onesv, acc):
        cid = lax.axis_index("c")
        sid = lax.axis_index("s")
        wid = sid * NC + cid
        ibase = pl.multiple_of(sid * RPT, 8)
        pltpu.sync_copy(ones_h.at[pl.ds(ibase, RPT)], acc.at[pl.ds(ibase, RPT)])

        @pl.when(sid == NS - 1)
        def _():
            tb = NS * RPT
            pltpu.sync_copy(ones_h.at[pl.ds(tb, TAIL)], acc.at[pl.ds(tb, TAIL)])

        pltpu.sync_copy(ones_h.at[pl.ds(0, IW)], onesv)
        plsc.subcore_barrier()

        def body(t, carry):
            rb = pl.multiple_of(wid * ROWS_PW + t * KR, 8)
            pltpu.sync_copy(dst_h.at[pl.ds(rb, KR)], dstv)
            for j in range(KR):
                pltpu.sync_copy(onesv, acc.at[dstv.at[j]], add=True)
            return carry

        lax.fori_loop(0, ITERS, body, 0)
        plsc.subcore_barrier()
        pltpu.sync_copy(acc.at[pl.ds(ibase, RPT)],
                        out_h.at[cid, pl.ds(ibase, RPT)])

        @pl.when(sid == NS - 1)
        def _():
            tb = NS * RPT
            pltpu.sync_copy(acc.at[pl.ds(tb, TAIL)],
                            out_h.at[cid, pl.ds(tb, TAIL)])

    return deg_kernel(dst2d, ones16)


def _edge_pipeline(src_h, dst_h, table, acc, srcv, dstv, rows, gsems, ssems,
                   isems, row_base, iters, sb):
    def fire_idx(t, slot):
        rb = pl.multiple_of(row_base(t), 8)
        pltpu.async_copy(src_h.at[pl.ds(rb, KR)],
                         srcv.at[pl.ds(slot * KR, KR)], isems[2 * slot])
        pltpu.async_copy(dst_h.at[pl.ds(rb, KR)],
                         dstv.at[pl.ds(slot * KR, KR)], isems[2 * slot + 1])

    def wait_idx(t, slot):
        rb = pl.multiple_of(row_base(t), 8)
        pltpu.make_async_copy(src_h.at[pl.ds(rb, KR)],
                              srcv.at[pl.ds(slot * KR, KR)],
                              isems[2 * slot]).wait()
        pltpu.make_async_copy(dst_h.at[pl.ds(rb, KR)],
                              dstv.at[pl.ds(slot * KR, KR)],
                              isems[2 * slot + 1]).wait()

    def process(slot):
        base = slot * KR
        gh, sh = {}, {}

        def fire_gather(j):
            s = j % sb
            gh[j] = pltpu.async_copy(table.at[srcv.at[base + j]],
                                     rows.at[pl.ds(s * IW, IW)], gsems[s])

        def fire_scatter(j):
            s = j % sb
            sh[j] = pltpu.async_copy(rows.at[pl.ds(s * IW, IW)],
                                     acc.at[dstv.at[base + j]], ssems[s],
                                     add=True)

        for k in range(min(sb, KR)):
            fire_gather(k)
        for j in range(KR):
            gh[j].wait()
            fire_scatter(j)
            k = j + sb
            if k < KR:
                sh[j].wait()
                fire_gather(k)
        for j in range(max(0, KR - sb), KR):
            sh[j].wait()

    def body(t, carry):
        rb = pl.multiple_of(row_base(t), 8)
        pltpu.sync_copy(src_h.at[pl.ds(rb, KR)], srcv.at[pl.ds(0, KR)])
        pltpu.sync_copy(dst_h.at[pl.ds(rb, KR)], dstv.at[pl.ds(0, KR)])
        process(0)
        return carry

    lax.fori_loop(0, iters, body, 0)


def _sc_scatter(hp, src2d, dst2d):
    F = hp.shape[1]
    table_in_spmem = F <= 64
    sb = 5 if table_in_spmem else SB
    mesh = plsc.VectorSubcoreMesh(core_axis_name="c", subcore_axis_name="s")

    scratch = [
        pltpu.VMEM((2 * KR, IW), jnp.int32),
        pltpu.VMEM((2 * KR, IW), jnp.int32),
        pltpu.VMEM((sb * IW, F), jnp.float32),
        pltpu.VMEM_SHARED((N_ACC, F), jnp.float32),
    ] + [pltpu.SemaphoreType.DMA] * (2 * sb + 4)
    if table_in_spmem:
        scratch.append(pltpu.VMEM_SHARED((N, F), jnp.float32))

    @functools.partial(
        pl.kernel,
        out_type=jax.ShapeDtypeStruct((2, N, F), jnp.float32),
        mesh=mesh,
        scratch_types=scratch,
        compiler_params=pltpu.CompilerParams(use_tc_tiling_on_sc=False),
    )
    def scatter_kernel(hp_h, src_h, dst_h, out_h, srcv, dstv, rows, acc,
                       *rest):
        cid = lax.axis_index("c")
        sid = lax.axis_index("s")
        wid = sid * NC + cid
        gsems = rest[:sb]
        ssems = rest[sb:2 * sb]
        isems = rest[2 * sb:2 * sb + 4]
        table = rest[2 * sb + 4] if table_in_spmem else hp_h
        ibase = pl.multiple_of(sid * RPT, 8)
        pltpu.sync_copy(hp_h.at[pl.ds(ibase, RPT)], acc.at[pl.ds(ibase, RPT)])
        if table_in_spmem:
            pltpu.sync_copy(hp_h.at[pl.ds(ibase, RPT)],
                            table.at[pl.ds(ibase, RPT)])

        @pl.when(sid == NS - 1)
        def _():
            tb = NS * RPT
            pltpu.sync_copy(hp_h.at[pl.ds(tb, TAIL)], acc.at[pl.ds(tb, TAIL)])
            if table_in_spmem:
                pltpu.sync_copy(hp_h.at[pl.ds(tb, TAIL)],
                                table.at[pl.ds(tb, TAIL)])

        plsc.subcore_barrier()

        _edge_pipeline(src_h, dst_h, table, acc, srcv, dstv, rows,
                       gsems, ssems, isems,
                       lambda t: wid * ROWS_PW + t * KR, ITERS, sb)
        plsc.subcore_barrier()
        pltpu.sync_copy(acc.at[pl.ds(ibase, RPT)],
                        out_h.at[cid, pl.ds(ibase, RPT)])

        @pl.when(sid == NS - 1)
        def _():
            tb = NS * RPT
            pltpu.sync_copy(acc.at[pl.ds(tb, TAIL)],
                            out_h.at[cid, pl.ds(tb, TAIL)])

    return scatter_kernel(hp, src2d, dst2d)


def _sc_scatter_panels(hpa, hpb, src2d, dst2d):
    HH = hpa.shape[1]
    sb = 5
    rows_pt = (E_PAD // IW) // NS
    iters = rows_pt // KR
    mesh = plsc.VectorSubcoreMesh(core_axis_name="c", subcore_axis_name="s")

    scratch = [
        pltpu.VMEM((2 * KR, IW), jnp.int32),
        pltpu.VMEM((2 * KR, IW), jnp.int32),
        pltpu.VMEM((sb * IW, HH), jnp.float32),
        pltpu.VMEM_SHARED((N_ACC, HH), jnp.float32),
        pltpu.VMEM_SHARED((N, HH), jnp.float32),
    ] + [pltpu.SemaphoreType.DMA] * (2 * sb + 4)

    @functools.partial(
        pl.kernel,
        out_type=jax.ShapeDtypeStruct((2, N, HH), jnp.float32),
        mesh=mesh,
        scratch_types=scratch,
        compiler_params=pltpu.CompilerParams(use_tc_tiling_on_sc=False),
    )
    def panel_kernel(hpa_h, hpb_h, src_h, dst_h, out_h, srcv, dstv, rows,
                     acc, table, *sems):
        cid = lax.axis_index("c")
        sid = lax.axis_index("s")
        gsems = sems[:sb]
        ssems = sems[sb:2 * sb]
        isems = sems[2 * sb:]
        ibase = pl.multiple_of(sid * RPT, 8)

        def init_from(hp_h):
            pltpu.sync_copy(hp_h.at[pl.ds(ibase, RPT)],
                            acc.at[pl.ds(ibase, RPT)])
            pltpu.sync_copy(hp_h.at[pl.ds(ibase, RPT)],
                            table.at[pl.ds(ibase, RPT)])

            @pl.when(sid == NS - 1)
            def _():
                tb = NS * RPT
                pltpu.sync_copy(hp_h.at[pl.ds(tb, TAIL)],
                                acc.at[pl.ds(tb, TAIL)])
                pltpu.sync_copy(hp_h.at[pl.ds(tb, TAIL)],
                                table.at[pl.ds(tb, TAIL)])

        @pl.when(cid == 0)
        def _():
            init_from(hpa_h)

        @pl.when(cid == 1)
        def _():
            init_from(hpb_h)

        plsc.subcore_barrier()

        _edge_pipeline(src_h, dst_h, table, acc, srcv, dstv, rows,
                       gsems, ssems, isems,
                       lambda t: sid * rows_pt + t * KR, iters, sb)
        plsc.subcore_barrier()
        pltpu.sync_copy(acc.at[pl.ds(ibase, RPT)],
                        out_h.at[cid, pl.ds(ibase, RPT)])

        @pl.when(sid == NS - 1)
        def _():
            tb = NS * RPT
            pltpu.sync_copy(acc.at[pl.ds(tb, TAIL)],
                            out_h.at[cid, pl.ds(tb, TAIL)])

    return panel_kernel(hpa, hpb, src2d, dst2d)


_RB = 1000


def _dis_from(deg_blk):
    deg = deg_blk[0, :, 0:1] + deg_blk[1, :, 0:1] - 1.0
    return lax.rsqrt(deg)


def _prep1_body(x_ref, w_ref, deg_ref, out_ref):
    dis = _dis_from(deg_ref[...])
    out_ref[...] = dis * jnp.dot(x_ref[...], w_ref[...],
                                 preferred_element_type=jnp.float32)


def _tc_prep1(x, W1, degout):
    DIN, H = W1.shape
    return pl.pallas_call(
        _prep1_body,
        grid=(N // _RB,),
        in_specs=[
            pl.BlockSpec((_RB, DIN), lambda i: (i, 0)),
            pl.BlockSpec((DIN, H), lambda i: (0, 0)),
            pl.BlockSpec((2, _RB, 16), lambda i: (0, i, 0)),
        ],
        out_specs=pl.BlockSpec((_RB, H), lambda i: (i, 0)),
        out_shape=jax.ShapeDtypeStruct((N, H), jnp.float32),
    )(x, W1, degout)


def _mid_body(p_ref, hp1_ref, deg_ref, b1_ref, w2_ref, outa_ref, outb_ref):
    dis = _dis_from(deg_ref[...])
    s = p_ref[0] + p_ref[1] - hp1_ref[...]
    h1 = jnp.maximum(dis * s + b1_ref[...], 0.0)
    hp2 = dis * jnp.dot(h1, w2_ref[...],
                        preferred_element_type=jnp.float32)
    hh = hp2.shape[1] // 2
    outa_ref[...] = hp2[:, :hh]
    outb_ref[...] = hp2[:, hh:]


def _tc_mid(p, hp1, degout, b1r, W2):
    H1, H2 = W2.shape
    HH = H2 // 2
    return pl.pallas_call(
        _mid_body,
        grid=(N // _RB,),
        in_specs=[
            pl.BlockSpec((2, _RB, H1), lambda i: (0, i, 0)),
            pl.BlockSpec((_RB, H1), lambda i: (i, 0)),
            pl.BlockSpec((2, _RB, 16), lambda i: (0, i, 0)),
            pl.BlockSpec((1, H1), lambda i: (0, 0)),
            pl.BlockSpec((H1, H2), lambda i: (0, 0)),
        ],
        out_specs=[
            pl.BlockSpec((_RB, HH), lambda i: (i, 0)),
            pl.BlockSpec((_RB, HH), lambda i: (i, 0)),
        ],
        out_shape=[
            jax.ShapeDtypeStruct((N, HH), jnp.float32),
            jax.ShapeDtypeStruct((N, HH), jnp.float32),
        ],
    )(p, hp1, degout, b1r, W2)


def _final_body(q_ref, deg_ref, b2_ref,
                batch_ref, wf_ref, bf_ref,
                wm1_ref, bm1_ref, wm2_ref, bm2_ref, wm3_ref, bm3_ref,
                wm4_ref, bm4_ref, y_ref, out_ref, loss_ref, sums_acc, cnt_acc):
    i = pl.program_id(0)

    @pl.when(i == 0)
    def _():
        sums_acc[...] = jnp.zeros_like(sums_acc)
        cnt_acc[...] = jnp.zeros_like(cnt_acc)

    dis = _dis_from(deg_ref[...])
    hh = q_ref.shape[2]
    h2a = jnp.maximum(dis * q_ref[0] + b2_ref[:, :hh], 0.0)
    h2b = jnp.maximum(dis * q_ref[1] + b2_ref[:, hh:], 0.0)
    h2 = jnp.concatenate([h2a, h2b], axis=1)
    b = batch_ref[...]
    onehot = (b == lax.broadcasted_iota(jnp.int32, (_RB, NG), 1)
              ).astype(jnp.float32)
    dn = (((0,), (0,)), ((), ()))
    sums_acc[...] += lax.dot_general(onehot, h2, dn,
                                     preferred_element_type=jnp.float32)
    cnt_acc[...] += lax.dot_general(onehot, jnp.ones((_RB, 1), jnp.float32),
                                    dn, preferred_element_type=jnp.float32)

    @pl.when(i == pl.num_programs(0) - 1)
    def _():
        pool = sums_acc[...] / jnp.maximum(cnt_acc[...], 1.0)
        g = jnp.dot(pool, wf_ref[...],
                    preferred_element_type=jnp.float32) + bf_ref[...]
        m = jnp.maximum(jnp.dot(g, wm1_ref[...],
                                preferred_element_type=jnp.float32)
                        + bm1_ref[...], 0.0)
        m = jnp.maximum(jnp.dot(m, wm2_ref[...],
                                preferred_element_type=jnp.float32)
                        + bm2_ref[...], 0.0)
        m = jnp.maximum(jnp.dot(m, wm3_ref[...],
                                preferred_element_type=jnp.float32)
                        + bm3_ref[...], 0.0)
        o = jnp.dot(m, wm4_ref[...],
                    preferred_element_type=jnp.float32) + bm4_ref[...]
        out_ref[...] = o
        loss_ref[...] = jnp.mean((o - y_ref[...]) ** 2).reshape(1, 1)


def _tc_final(q, degout, b2r, batch2d, Wf, bfr,
              Wm1, bm1r, Wm2, bm2r, Wm3, bm3r, Wm4, bm4r, y):
    HH = q.shape[2]
    H2 = 2 * HH
    D = Wf.shape[1]

    def full(shp):
        nd = len(shp)
        return pl.BlockSpec(shp, lambda i, _n=nd: (0,) * _n)

    return pl.pallas_call(
        _final_body,
        grid=(N // _RB,),
        in_specs=[
            pl.BlockSpec((2, _RB, HH), lambda i: (0, i, 0)),
            pl.BlockSpec((2, _RB, 16), lambda i: (0, i, 0)),
            full((1, H2)),
            pl.BlockSpec((_RB, 1), lambda i: (i, 0)),
            full((H2, D)),
            full((1, D)),
            full((D, 32)), full((1, 32)),
            full((32, 16)), full((1, 16)),
            full((16, 8)), full((1, 8)),
            full((8, 1)), full((1, 1)),
            full((NG, 1)),
        ],
        out_specs=[
            pl.BlockSpec((NG, 1), lambda i: (0, 0)),
            pl.BlockSpec((1, 1), lambda i: (0, 0)),
        ],
        out_shape=[
            jax.ShapeDtypeStruct((NG, 1), jnp.float32),
            jax.ShapeDtypeStruct((1, 1), jnp.float32),
        ],
        scratch_shapes=[
            pltpu.VMEM((NG, H2), jnp.float32),
            pltpu.VMEM((NG, 1), jnp.float32),
        ],
    )(q, degout, b2r, batch2d, Wf, bfr, Wm1, bm1r, Wm2, bm2r,
      Wm3, bm3r, Wm4, bm4r, y)


def kernel(x, edge_index, batch, y, W1, b1, W2, b2, Wf, bf,
           Wm1, bm1, Wm2, bm2, Wm3, bm3, Wm4, bm4):
    npad = E_PAD - E
    src_pad = jnp.concatenate(
        [edge_index[0], jnp.zeros((npad,), jnp.int32)])
    dst_pad = jnp.concatenate(
        [edge_index[1], N + (jnp.arange(npad, dtype=jnp.int32) % PAD_ROWS)])
    src2d = src_pad.reshape(E_PAD // IW, IW)
    dst2d = dst_pad.reshape(E_PAD // IW, IW)
    ones16 = jnp.ones((N, 16), jnp.float32)

    degout = _sc_degree(dst2d, ones16)
    hp1 = _tc_prep1(x, W1, degout)
    p = _sc_scatter(hp1, src2d, dst2d)
    hp2a, hp2b = _tc_mid(p, hp1, degout, b1.reshape(1, -1), W2)
    q = _sc_scatter_panels(hp2a, hp2b, src2d, dst2d)
    out, loss = _tc_final(
        q, degout, b2.reshape(1, -1), batch.reshape(-1, 1),
        Wf, bf.reshape(1, -1), Wm1, bm1.reshape(1, -1), Wm2,
        bm2.reshape(1, -1), Wm3, bm3.reshape(1, -1), Wm4,
        bm4.reshape(1, -1), y)
    return out, loss.reshape(())


# speedup vs baseline: 1.1043x; 1.0216x over previous
import functools

import jax
import jax.numpy as jnp
from jax import lax
from jax.experimental import pallas as pl
from jax.experimental.pallas import tpu as pltpu
from jax.experimental.pallas import tpu_sc as plsc

N = 10000
E = 320000
NG = 64

NC = 2
NS = 16
NW = NC * NS
IW = 128
KR = 8
CHUNK = KR * IW
E_PAD = 327680
ITERS = E_PAD // (NW * CHUNK)
ROWS_PW = E_PAD // IW // NW
PAD_ROWS = 128
N_ACC = N + PAD_ROWS
SB = 2

RPT = 624
TAIL = N - NS * RPT


def _sc_degree(dst2d, ones16):
    mesh = plsc.VectorSubcoreMesh(core_axis_name="c", subcore_axis_name="s")

    @functools.partial(
        pl.kernel,
        out_type=jax.ShapeDtypeStruct((2, N, 16), jnp.float32),
        mesh=mesh,
        scratch_types=[
            pltpu.VMEM((2 * KR, IW), jnp.int32),
            pltpu.VMEM((IW, 16), jnp.float32),
            pltpu.VMEM_SHARED((N_ACC, 16), jnp.float32),
            pltpu.SemaphoreType.DMA,
            pltpu.SemaphoreType.DMA,
        ],
        compiler_params=pltpu.CompilerParams(use_tc_tiling_on_sc=False),
    )
    def deg_kernel(dst_h, ones_h, out_h, dstv, onesv, acc, semA, semB):
        cid = lax.axis_index("c")
        sid = lax.axis_index("s")
        wid = sid * NC + cid
        ibase = pl.multiple_of(sid * RPT, 8)
        pltpu.sync_copy(ones_h.at[pl.ds(ibase, RPT)], acc.at[pl.ds(ibase, RPT)])

        @pl.when(sid == NS - 1)
        def _():
            tb = NS * RPT
            pltpu.sync_copy(ones_h.at[pl.ds(tb, TAIL)], acc.at[pl.ds(tb, TAIL)])

        pltpu.sync_copy(ones_h.at[pl.ds(0, IW)], onesv)
        plsc.subcore_barrier()

        def body(t2, carry):
            ha, hb = [], []
            rba = pl.multiple_of(wid * ROWS_PW + (2 * t2) * KR, 8)
            pltpu.sync_copy(dst_h.at[pl.ds(rba, KR)], dstv.at[pl.ds(0, KR)])
            for j in range(KR):
                ha.append(pltpu.async_copy(onesv, acc.at[dstv.at[j]],
                                           semA, add=True))
            rbb = pl.multiple_of(wid * ROWS_PW + (2 * t2 + 1) * KR, 8)
            pltpu.sync_copy(dst_h.at[pl.ds(rbb, KR)], dstv.at[pl.ds(KR, KR)])
            for j in range(KR):
                hb.append(pltpu.async_copy(onesv, acc.at[dstv.at[KR + j]],
                                           semB, add=True))
            for h in ha + hb:
                h.wait()
            return carry

        lax.fori_loop(0, ITERS // 2, body, 0)
        plsc.subcore_barrier()
        pltpu.sync_copy(acc.at[pl.ds(ibase, RPT)],
                        out_h.at[cid, pl.ds(ibase, RPT)])

        @pl.when(sid == NS - 1)
        def _():
            tb = NS * RPT
            pltpu.sync_copy(acc.at[pl.ds(tb, TAIL)],
                            out_h.at[cid, pl.ds(tb, TAIL)])

    return deg_kernel(dst2d, ones16)


def _edge_pipeline(src_h, dst_h, table, acc, srcv, dstv, rows, gsems, ssems,
                   isems, row_base, iters, sb):
    def fire_idx(t, slot):
        rb = pl.multiple_of(row_base(t), 8)
        pltpu.async_copy(src_h.at[pl.ds(rb, KR)],
                         srcv.at[pl.ds(slot * KR, KR)], isems[2 * slot])
        pltpu.async_copy(dst_h.at[pl.ds(rb, KR)],
                         dstv.at[pl.ds(slot * KR, KR)], isems[2 * slot + 1])

    def wait_idx(t, slot):
        rb = pl.multiple_of(row_base(t), 8)
        pltpu.make_async_copy(src_h.at[pl.ds(rb, KR)],
                              srcv.at[pl.ds(slot * KR, KR)],
                              isems[2 * slot]).wait()
        pltpu.make_async_copy(dst_h.at[pl.ds(rb, KR)],
                              dstv.at[pl.ds(slot * KR, KR)],
                              isems[2 * slot + 1]).wait()

    def process(slot):
        base = slot * KR
        gh, sh = {}, {}

        def fire_gather(j):
            s = j % sb
            gh[j] = pltpu.async_copy(table.at[srcv.at[base + j]],
                                     rows.at[pl.ds(s * IW, IW)], gsems[s])

        def fire_scatter(j):
            s = j % sb
            sh[j] = pltpu.async_copy(rows.at[pl.ds(s * IW, IW)],
                                     acc.at[dstv.at[base + j]], ssems[s],
                                     add=True)

        for k in range(min(sb, KR)):
            fire_gather(k)
        for j in range(KR):
            gh[j].wait()
            fire_scatter(j)
            k = j + sb
            if k < KR:
                sh[j].wait()
                fire_gather(k)
        for j in range(max(0, KR - sb), KR):
            sh[j].wait()

    def body(t, carry):
        rb = pl.multiple_of(row_base(t), 8)
        pltpu.sync_copy(src_h.at[pl.ds(rb, KR)], srcv.at[pl.ds(0, KR)])
        pltpu.sync_copy(dst_h.at[pl.ds(rb, KR)], dstv.at[pl.ds(0, KR)])
        process(0)
        return carry

    lax.fori_loop(0, iters, body, 0)


def _sc_scatter(hp, zeros, src2d, dst2d):
    F = hp.shape[1]
    table_in_spmem = F <= 64
    sb = 5 if table_in_spmem else SB
    mesh = plsc.VectorSubcoreMesh(core_axis_name="c", subcore_axis_name="s")

    scratch = [
        pltpu.VMEM((2 * KR, IW), jnp.int32),
        pltpu.VMEM((2 * KR, IW), jnp.int32),
        pltpu.VMEM((sb * IW, F), jnp.float32),
        pltpu.VMEM_SHARED((N_ACC, F), jnp.float32),
    ] + [pltpu.SemaphoreType.DMA] * (2 * sb + 4)
    if table_in_spmem:
        scratch.append(pltpu.VMEM_SHARED((N, F), jnp.float32))

    @functools.partial(
        pl.kernel,
        out_type=jax.ShapeDtypeStruct((2, N, F), jnp.float32),
        mesh=mesh,
        scratch_types=scratch,
        compiler_params=pltpu.CompilerParams(use_tc_tiling_on_sc=False),
    )
    def scatter_kernel(hp_h, zero_h, src_h, dst_h, out_h, srcv, dstv, rows,
                       acc, *rest):
        cid = lax.axis_index("c")
        sid = lax.axis_index("s")
        wid = sid * NC + cid
        gsems = rest[:sb]
        ssems = rest[sb:2 * sb]
        isems = rest[2 * sb:2 * sb + 4]
        table = rest[2 * sb + 4] if table_in_spmem else hp_h
        ibase = pl.multiple_of(sid * RPT, 8)

        def init_acc(src):
            pltpu.sync_copy(src.at[pl.ds(ibase, RPT)],
                            acc.at[pl.ds(ibase, RPT)])

            @pl.when(sid == NS - 1)
            def _():
                tb = NS * RPT
                pltpu.sync_copy(src.at[pl.ds(tb, TAIL)],
                                acc.at[pl.ds(tb, TAIL)])

        @pl.when(cid == 0)
        def _():
            init_acc(hp_h)

        @pl.when(cid == 1)
        def _():
            init_acc(zero_h)

        if table_in_spmem:
            pltpu.sync_copy(hp_h.at[pl.ds(ibase, RPT)],
                            table.at[pl.ds(ibase, RPT)])

            @pl.when(sid == NS - 1)
            def _():
                tb = NS * RPT
                pltpu.sync_copy(hp_h.at[pl.ds(tb, TAIL)],
                                table.at[pl.ds(tb, TAIL)])

        plsc.subcore_barrier()

        _edge_pipeline(src_h, dst_h, table, acc, srcv, dstv, rows,
                       gsems, ssems, isems,
                       lambda t: wid * ROWS_PW + t * KR, ITERS, sb)
        plsc.subcore_barrier()
        pltpu.sync_copy(acc.at[pl.ds(ibase, RPT)],
                        out_h.at[cid, pl.ds(ibase, RPT)])

        @pl.when(sid == NS - 1)
        def _():
            tb = NS * RPT
            pltpu.sync_copy(acc.at[pl.ds(tb, TAIL)],
                            out_h.at[cid, pl.ds(tb, TAIL)])

    return scatter_kernel(hp, zeros, src2d, dst2d)


def _sc_scatter_panels(hpa, hpb, src2d, dst2d):
    HH = hpa.shape[1]
    sb = 5
    rows_pt = (E_PAD // IW) // NS
    iters = rows_pt // KR
    mesh = plsc.VectorSubcoreMesh(core_axis_name="c", subcore_axis_name="s")

    scratch = [
        pltpu.VMEM((2 * KR, IW), jnp.int32),
        pltpu.VMEM((2 * KR, IW), jnp.int32),
        pltpu.VMEM((sb * IW, HH), jnp.float32),
        pltpu.VMEM_SHARED((N_ACC, HH), jnp.float32),
        pltpu.VMEM_SHARED((N, HH), jnp.float32),
    ] + [pltpu.SemaphoreType.DMA] * (2 * sb + 4)

    @functools.partial(
        pl.kernel,
        out_type=jax.ShapeDtypeStruct((2, N, HH), jnp.float32),
        mesh=mesh,
        scratch_types=scratch,
        compiler_params=pltpu.CompilerParams(use_tc_tiling_on_sc=False),
    )
    def panel_kernel(hpa_h, hpb_h, src_h, dst_h, out_h, srcv, dstv, rows,
                     acc, table, *sems):
        cid = lax.axis_index("c")
        sid = lax.axis_index("s")
        gsems = sems[:sb]
        ssems = sems[sb:2 * sb]
        isems = sems[2 * sb:]
        ibase = pl.multiple_of(sid * RPT, 8)

        def init_from(hp_h):
            pltpu.sync_copy(hp_h.at[pl.ds(ibase, RPT)],
                            acc.at[pl.ds(ibase, RPT)])
            pltpu.sync_copy(hp_h.at[pl.ds(ibase, RPT)],
                            table.at[pl.ds(ibase, RPT)])

            @pl.when(sid == NS - 1)
            def _():
                tb = NS * RPT
                pltpu.sync_copy(hp_h.at[pl.ds(tb, TAIL)],
                                acc.at[pl.ds(tb, TAIL)])
                pltpu.sync_copy(hp_h.at[pl.ds(tb, TAIL)],
                                table.at[pl.ds(tb, TAIL)])

        @pl.when(cid == 0)
        def _():
            init_from(hpa_h)

        @pl.when(cid == 1)
        def _():
            init_from(hpb_h)

        plsc.subcore_barrier()

        _edge_pipeline(src_h, dst_h, table, acc, srcv, dstv, rows,
                       gsems, ssems, isems,
                       lambda t: sid * rows_pt + t * KR, iters, sb)
        plsc.subcore_barrier()
        pltpu.sync_copy(acc.at[pl.ds(ibase, RPT)],
                        out_h.at[cid, pl.ds(ibase, RPT)])

        @pl.when(sid == NS - 1)
        def _():
            tb = NS * RPT
            pltpu.sync_copy(acc.at[pl.ds(tb, TAIL)],
                            out_h.at[cid, pl.ds(tb, TAIL)])

    return panel_kernel(hpa, hpb, src2d, dst2d)


_RB = 1000


def _dis_from(deg_blk):
    deg = deg_blk[0, :, 0:1] + deg_blk[1, :, 0:1] - 1.0
    return lax.rsqrt(deg)


def _prep1_body(x_ref, w_ref, deg_ref, out_ref):
    dis = _dis_from(deg_ref[...])
    out_ref[...] = dis * jnp.dot(x_ref[...], w_ref[...],
                                 preferred_element_type=jnp.float32)


def _tc_prep1(x, W1, degout):
    DIN, H = W1.shape
    return pl.pallas_call(
        _prep1_body,
        grid=(N // _RB,),
        in_specs=[
            pl.BlockSpec((_RB, DIN), lambda i: (i, 0)),
            pl.BlockSpec((DIN, H), lambda i: (0, 0)),
            pl.BlockSpec((2, _RB, 16), lambda i: (0, i, 0)),
        ],
        out_specs=pl.BlockSpec((_RB, H), lambda i: (i, 0)),
        out_shape=jax.ShapeDtypeStruct((N, H), jnp.float32),
    )(x, W1, degout)


def _mid_body(p_ref, deg_ref, b1_ref, w2_ref, outa_ref, outb_ref):
    dis = _dis_from(deg_ref[...])
    s = p_ref[0] + p_ref[1]
    h1 = jnp.maximum(dis * s + b1_ref[...], 0.0)
    hp2 = dis * jnp.dot(h1, w2_ref[...],
                        preferred_element_type=jnp.float32)
    hh = hp2.shape[1] // 2
    outa_ref[...] = hp2[:, :hh]
    outb_ref[...] = hp2[:, hh:]


def _tc_mid(p, degout, b1r, W2):
    H1, H2 = W2.shape
    HH = H2 // 2
    return pl.pallas_call(
        _mid_body,
        grid=(N // _RB,),
        in_specs=[
            pl.BlockSpec((2, _RB, H1), lambda i: (0, i, 0)),
            pl.BlockSpec((2, _RB, 16), lambda i: (0, i, 0)),
            pl.BlockSpec((1, H1), lambda i: (0, 0)),
            pl.BlockSpec((H1, H2), lambda i: (0, 0)),
        ],
        out_specs=[
            pl.BlockSpec((_RB, HH), lambda i: (i, 0)),
            pl.BlockSpec((_RB, HH), lambda i: (i, 0)),
        ],
        out_shape=[
            jax.ShapeDtypeStruct((N, HH), jnp.float32),
            jax.ShapeDtypeStruct((N, HH), jnp.float32),
        ],
    )(p, degout, b1r, W2)


def _final_body(q_ref, deg_ref, b2_ref,
                batch_ref, wf_ref, bf_ref,
                wm1_ref, bm1_ref, wm2_ref, bm2_ref, wm3_ref, bm3_ref,
                wm4_ref, bm4_ref, y_ref, out_ref, loss_ref, sums_acc, cnt_acc):
    i = pl.program_id(0)

    @pl.when(i == 0)
    def _():
        sums_acc[...] = jnp.zeros_like(sums_acc)
        cnt_acc[...] = jnp.zeros_like(cnt_acc)

    dis = _dis_from(deg_ref[...])
    hh = q_ref.shape[2]
    h2a = jnp.maximum(dis * q_ref[0] + b2_ref[:, :hh], 0.0)
    h2b = jnp.maximum(dis * q_ref[1] + b2_ref[:, hh:], 0.0)
    h2 = jnp.concatenate([h2a, h2b], axis=1)
    b = batch_ref[...]
    onehot = (b == lax.broadcasted_iota(jnp.int32, (_RB, NG), 1)
              ).astype(jnp.float32)
    dn = (((0,), (0,)), ((), ()))
    sums_acc[...] += lax.dot_general(onehot, h2, dn,
                                     preferred_element_type=jnp.float32)
    cnt_acc[...] += lax.dot_general(onehot, jnp.ones((_RB, 1), jnp.float32),
                                    dn, preferred_element_type=jnp.float32)

    @pl.when(i == pl.num_programs(0) - 1)
    def _():
        pool = sums_acc[...] / jnp.maximum(cnt_acc[...], 1.0)
        g = jnp.dot(pool, wf_ref[...],
                    preferred_element_type=jnp.float32) + bf_ref[...]
        m = jnp.maximum(jnp.dot(g, wm1_ref[...],
                                preferred_element_type=jnp.float32)
                        + bm1_ref[...], 0.0)
        m = jnp.maximum(jnp.dot(m, wm2_ref[...],
                                preferred_element_type=jnp.float32)
                        + bm2_ref[...], 0.0)
        m = jnp.maximum(jnp.dot(m, wm3_ref[...],
                                preferred_element_type=jnp.float32)
                        + bm3_ref[...], 0.0)
        o = jnp.dot(m, wm4_ref[...],
                    preferred_element_type=jnp.float32) + bm4_ref[...]
        out_ref[...] = o
        loss_ref[...] = jnp.mean((o - y_ref[...]) ** 2).reshape(1, 1)


def _tc_final(q, degout, b2r, batch2d, Wf, bfr,
              Wm1, bm1r, Wm2, bm2r, Wm3, bm3r, Wm4, bm4r, y):
    HH = q.shape[2]
    H2 = 2 * HH
    D = Wf.shape[1]

    def full(shp):
        nd = len(shp)
        return pl.BlockSpec(shp, lambda i, _n=nd: (0,) * _n)

    return pl.pallas_call(
        _final_body,
        grid=(N // _RB,),
        in_specs=[
            pl.BlockSpec((2, _RB, HH), lambda i: (0, i, 0)),
            pl.BlockSpec((2, _RB, 16), lambda i: (0, i, 0)),
            full((1, H2)),
            pl.BlockSpec((_RB, 1), lambda i: (i, 0)),
            full((H2, D)),
            full((1, D)),
            full((D, 32)), full((1, 32)),
            full((32, 16)), full((1, 16)),
            full((16, 8)), full((1, 8)),
            full((8, 1)), full((1, 1)),
            full((NG, 1)),
        ],
        out_specs=[
            pl.BlockSpec((NG, 1), lambda i: (0, 0)),
            pl.BlockSpec((1, 1), lambda i: (0, 0)),
        ],
        out_shape=[
            jax.ShapeDtypeStruct((NG, 1), jnp.float32),
            jax.ShapeDtypeStruct((1, 1), jnp.float32),
        ],
        scratch_shapes=[
            pltpu.VMEM((NG, H2), jnp.float32),
            pltpu.VMEM((NG, 1), jnp.float32),
        ],
    )(q, degout, b2r, batch2d, Wf, bfr, Wm1, bm1r, Wm2, bm2r,
      Wm3, bm3r, Wm4, bm4r, y)


def kernel(x, edge_index, batch, y, W1, b1, W2, b2, Wf, bf,
           Wm1, bm1, Wm2, bm2, Wm3, bm3, Wm4, bm4):
    npad = E_PAD - E
    src_pad = jnp.concatenate(
        [edge_index[0], jnp.zeros((npad,), jnp.int32)])
    dst_pad = jnp.concatenate(
        [edge_index[1], N + (jnp.arange(npad, dtype=jnp.int32) % PAD_ROWS)])
    src2d = src_pad.reshape(E_PAD // IW, IW)
    dst2d = dst_pad.reshape(E_PAD // IW, IW)
    ones16 = jnp.ones((N, 16), jnp.float32)

    degout = _sc_degree(dst2d, ones16)
    hp1 = _tc_prep1(x, W1, degout)
    zeros64 = jnp.zeros((N, W1.shape[1]), jnp.float32)
    p = _sc_scatter(hp1, zeros64, src2d, dst2d)
    hp2a, hp2b = _tc_mid(p, degout, b1.reshape(1, -1), W2)
    q = _sc_scatter_panels(hp2a, hp2b, src2d, dst2d)
    out, loss = _tc_final(
        q, degout, b2.reshape(1, -1), batch.reshape(-1, 1),
        Wf, bf.reshape(1, -1), Wm1, bm1.reshape(1, -1), Wm2,
        bm2.reshape(1, -1), Wm3, bm3.reshape(1, -1), Wm4,
        bm4.reshape(1, -1), y)
    return out, loss.reshape(())
